# Initial kernel scaffold; baseline (speedup 1.0000x reference)
import jax
import jax.numpy as jnp
from jax.experimental import pallas as pl


def kernel(atomic_num, degree, formal_charge, chirality, num_h, hybridization, E_atomic_num, E_degree, E_formal_charge, E_chirality, E_num_h, E_hybridization, W, b):
    raise NotImplementedError("write your pallas kernel here")


# speedup vs baseline: 9.8043x; 9.8043x over previous
import functools

import jax
import jax.numpy as jnp
from jax import lax
from jax.experimental import pallas as pl
from jax.experimental.pallas import tpu as pltpu
from jax.experimental.pallas import tpu_sc as plsc

N = 100000
ED = 64
OD = 128
VOCABS = (119, 11, 11, 5, 9, 8)
OFF = (0, 119, 130, 141, 146, 155)
VTOT = 163
VPAD = 256

T1_ROWS = 119 * 11
T1_PAD = 1312
T2_ROWS = 11 * 5 * 9 * 8
T2_PAD = 3968

NW = 32
PER_W = 3200
NPAD = NW * PER_W
CH = 128
NCH = PER_W // CH
TAIL = N % CH


def _table_build(ecat_ref, wt_ref, b_ref, t1_ref, t2_ref):
    tp = jnp.dot(ecat_ref[...], wt_ref[...], preferred_element_type=jnp.float32)
    r1 = lax.broadcasted_iota(jnp.int32, (T1_PAD, VPAD), 0)
    c1 = lax.broadcasted_iota(jnp.int32, (T1_PAD, VPAD), 1)
    s1 = jnp.where((c1 == r1 // 11) | (c1 == OFF[1] + r1 % 11), 1.0, 0.0)
    t1_ref[...] = jnp.dot(s1, tp, preferred_element_type=jnp.float32) + b_ref[...]
    r2 = lax.broadcasted_iota(jnp.int32, (T2_PAD, VPAD), 0)
    c2 = lax.broadcasted_iota(jnp.int32, (T2_PAD, VPAD), 1)
    hit = (
        (c2 == OFF[2] + r2 // 360)
        | (c2 == OFF[3] + (r2 // 72) % 5)
        | (c2 == OFF[4] + (r2 // 8) % 9)
        | (c2 == OFF[5] + r2 % 8)
    )
    s2 = jnp.where(hit, 1.0, 0.0)
    t2_ref[...] = jnp.dot(s2, tp, preferred_element_type=jnp.float32)


def _sc_body(idx_ref, t1_ref, t2_ref, out_ref,
             idx6, idx1, idx2, buf1, buf2, sem1, sem2):
    w = lax.axis_index("s") * 2 + lax.axis_index("c")
    base = w * PER_W
    pltpu.sync_copy(idx_ref.at[:, pl.ds(base, PER_W)], idx6)

    def chunk(j, carry):
        cb = j * CH
        row0 = base + cb

        @pl.when(row0 < N)
        def _():
            for i in range(CH // 16):
                s = i * 16
                sl = pl.ds(cb + s, 16)
                va = idx6[0, sl]
                vd = idx6[1, sl]
                idx1[pl.ds(s, 16)] = va * 11 + vd
                vfc = idx6[2, sl]
                vch = idx6[3, sl]
                vnh = idx6[4, sl]
                vhy = idx6[5, sl]
                idx2[pl.ds(s, 16)] = ((vfc * 5 + vch) * 9 + vnh) * 8 + vhy
            cp1 = pltpu.async_copy(t1_ref.at[idx1], buf1, sem1)
            cp2 = pltpu.async_copy(t2_ref.at[idx2], buf2, sem2)
            cp1.wait()
            cp2.wait()

            def addrow(r, c):
                for ci in range(OD // 16):
                    sl2 = pl.ds(ci * 16, 16)
                    buf1[r, sl2] = buf1[r, sl2] + buf2[r, sl2]
                return c

            lax.fori_loop(0, CH, addrow, 0)

            @pl.when(row0 + CH <= N)
            def _full():
                pltpu.sync_copy(buf1, out_ref.at[pl.ds(row0, CH)])

            @pl.when(row0 + CH > N)
            def _tail():
                pltpu.sync_copy(buf1.at[pl.ds(0, TAIL)],
                                out_ref.at[pl.ds(row0, TAIL)])

        return carry

    lax.fori_loop(0, NCH, chunk, 0)


@functools.cache
def _get_sc_call():
    return pl.kernel(
        _sc_body,
        out_type=jax.ShapeDtypeStruct((N, OD), jnp.float32),
        mesh=plsc.VectorSubcoreMesh(core_axis_name="c", subcore_axis_name="s"),
        scratch_types=[
            pltpu.VMEM((6, PER_W), jnp.int32),
            pltpu.VMEM((CH,), jnp.int32),
            pltpu.VMEM((CH,), jnp.int32),
            pltpu.VMEM((CH, OD), jnp.float32),
            pltpu.VMEM((CH, OD), jnp.float32),
            pltpu.SemaphoreType.DMA,
            pltpu.SemaphoreType.DMA,
        ],
    )


@jax.jit
def kernel(atomic_num, degree, formal_charge, chirality, num_h, hybridization,
           E_atomic_num, E_degree, E_formal_charge, E_chirality, E_num_h,
           E_hybridization, W, b):
    tables = (E_atomic_num, E_degree, E_formal_charge, E_chirality, E_num_h,
              E_hybridization)
    blocks = []
    for f, (e, v) in enumerate(zip(tables, VOCABS)):
        blocks.append(jnp.pad(e, ((0, 0), (f * ED, (5 - f) * ED))))
    ecat = jnp.concatenate(blocks, axis=0)
    ecat = jnp.pad(ecat, ((0, VPAD - VTOT), (0, 0)))

    t1, t2 = pl.pallas_call(
        _table_build,
        out_shape=[
            jax.ShapeDtypeStruct((T1_PAD, OD), jnp.float32),
            jax.ShapeDtypeStruct((T2_PAD, OD), jnp.float32),
        ],
    )(ecat, W.T, b.reshape(1, OD))

    idx = jnp.stack((atomic_num, degree, formal_charge, chirality, num_h,
                     hybridization))
    idx = jnp.pad(idx, ((0, 0), (0, NPAD - N)))
    return _get_sc_call()(idx, t1, t2)


# speedup vs baseline: 14.6419x; 1.4934x over previous
import functools

import jax
import jax.numpy as jnp
from jax import lax
from jax.experimental import pallas as pl
from jax.experimental.pallas import tpu as pltpu
from jax.experimental.pallas import tpu_sc as plsc

N = 100000
ED = 64
OD = 128
VOCABS = (119, 11, 11, 5, 9, 8)
OFF = (0, 119, 130, 141, 146, 155)
VTOT = 163
VPAD = 256

T1_PAD = 1312
T2_PAD = 3968

NW = 32
PER_W = 3200
CH = 128
NCH = PER_W // CH
NBUF = 3


def _table_build(ecat_ref, wt_ref, b_ref, t1_ref, t2_ref):
    tp = jnp.dot(ecat_ref[...], wt_ref[...], preferred_element_type=jnp.float32)
    r1 = lax.broadcasted_iota(jnp.int32, (T1_PAD, VPAD), 0)
    c1 = lax.broadcasted_iota(jnp.int32, (T1_PAD, VPAD), 1)
    s1 = jnp.where((c1 == r1 // 11) | (c1 == OFF[1] + r1 % 11), 1.0, 0.0)
    t1_ref[...] = jnp.dot(s1, tp, preferred_element_type=jnp.float32) + b_ref[...]
    r2 = lax.broadcasted_iota(jnp.int32, (T2_PAD, VPAD), 0)
    c2 = lax.broadcasted_iota(jnp.int32, (T2_PAD, VPAD), 1)
    hit = (
        (c2 == OFF[2] + r2 // 360)
        | (c2 == OFF[3] + (r2 // 72) % 5)
        | (c2 == OFF[4] + (r2 // 8) % 9)
        | (c2 == OFF[5] + r2 % 8)
    )
    s2 = jnp.where(hit, 1.0, 0.0)
    t2_ref[...] = jnp.dot(s2, tp, preferred_element_type=jnp.float32)


def _sc_body(a_ref, d_ref, fc_ref, ch_ref, nh_ref, hy_ref, t1_ref, t2_ref,
             out_ref, i0, i1, i2, i3, i4, i5, idx1a, idx2a, buf1, buf2,
             sg0, sg1, sg2, sw0, sw1, sw2):
    semg = (sg0, sg1, sg2)
    semw = (sw0, sw1, sw2)
    idx6 = (i0, i1, i2, i3, i4, i5)
    w = lax.axis_index("s") * 2 + lax.axis_index("c")
    base = lax.min(w * PER_W, N - PER_W)

    stage = [
        pltpu.async_copy(r.at[pl.ds(base, PER_W)], idx6[f], sg0)
        for f, r in enumerate((a_ref, d_ref, fc_ref, ch_ref, nh_ref, hy_ref))
    ]
    for cp in stage:
        cp.wait()

    def comb(i, c):
        sl = pl.ds(i * 16, 16)
        idx1a[sl] = i0[sl] * 11 + i1[sl]
        idx2a[sl] = ((i2[sl] * 5 + i3[sl]) * 9 + i4[sl]) * 8 + i5[sl]
        return c

    lax.fori_loop(0, PER_W // 16, comb, 0)

    def issue(j, s):
        cb = j * CH
        c1 = pltpu.async_copy(
            t1_ref.at[idx1a.at[pl.ds(cb, CH)]], buf1.at[s], semg[s])
        c2 = pltpu.async_copy(
            t2_ref.at[idx2a.at[pl.ds(cb, CH)]], buf2.at[s], semg[s])
        return c1, c2

    def add(s):
        def addrow(r, c):
            for ci in range(OD // 16):
                sl2 = pl.ds(ci * 16, 16)
                buf2[s, r, sl2] = buf2[s, r, sl2] + buf1[s, r, sl2]
            return c

        lax.fori_loop(0, CH, addrow, 0)

    cps = [None] * NBUF
    wrs = [None] * NBUF
    cps[0] = issue(0, 0)
    for j in range(NCH):
        s = j % NBUF
        if j + 1 < NCH:
            ns = (j + 1) % NBUF
            if wrs[ns] is not None:
                wrs[ns].wait()
                wrs[ns] = None
            cps[ns] = issue(j + 1, ns)
        cps[s][0].wait()
        cps[s][1].wait()
        add(s)
        wrs[s] = pltpu.async_copy(
            buf2.at[s], out_ref.at[pl.ds(base + j * CH, CH)], semw[s])
    for s in range(NBUF):
        if wrs[s] is not None:
            wrs[s].wait()


@functools.cache
def _get_sc_call():
    return pl.kernel(
        _sc_body,
        out_type=jax.ShapeDtypeStruct((N, OD), jnp.float32),
        mesh=plsc.VectorSubcoreMesh(core_axis_name="c", subcore_axis_name="s"),
        scratch_types=[
            pltpu.VMEM((PER_W,), jnp.int32),
            pltpu.VMEM((PER_W,), jnp.int32),
            pltpu.VMEM((PER_W,), jnp.int32),
            pltpu.VMEM((PER_W,), jnp.int32),
            pltpu.VMEM((PER_W,), jnp.int32),
            pltpu.VMEM((PER_W,), jnp.int32),
            pltpu.VMEM((PER_W,), jnp.int32),
            pltpu.VMEM((PER_W,), jnp.int32),
            pltpu.VMEM((NBUF, CH, OD), jnp.float32),
            pltpu.VMEM((NBUF, CH, OD), jnp.float32),
            pltpu.SemaphoreType.DMA,
            pltpu.SemaphoreType.DMA,
            pltpu.SemaphoreType.DMA,
            pltpu.SemaphoreType.DMA,
            pltpu.SemaphoreType.DMA,
            pltpu.SemaphoreType.DMA,
        ],
    )


@jax.jit
def kernel(atomic_num, degree, formal_charge, chirality, num_h, hybridization,
           E_atomic_num, E_degree, E_formal_charge, E_chirality, E_num_h,
           E_hybridization, W, b):
    tables = (E_atomic_num, E_degree, E_formal_charge, E_chirality, E_num_h,
              E_hybridization)
    blocks = [jnp.pad(e, ((0, 0), (f * ED, (5 - f) * ED)))
              for f, e in enumerate(tables)]
    ecat = jnp.concatenate(blocks, axis=0)
    ecat = jnp.pad(ecat, ((0, VPAD - VTOT), (0, 0)))

    t1, t2 = pl.pallas_call(
        _table_build,
        out_shape=[
            jax.ShapeDtypeStruct((T1_PAD, OD), jnp.float32),
            jax.ShapeDtypeStruct((T2_PAD, OD), jnp.float32),
        ],
    )(ecat, W.T, b.reshape(1, OD))

    return _get_sc_call()(atomic_num, degree, formal_charge, chirality,
                          num_h, hybridization, t1, t2)
